# baseline (speedup 1.0000x reference)
import functools

import jax
import jax.numpy as jnp
import numpy as np
from jax import lax
from jax.experimental import pallas as pl
from jax.experimental.pallas import tpu as pltpu
from jax.experimental.pallas import tpu_sc as plsc

N = 10000
E = 320000
D = 128
H = 128
G = 64
NC = 2

NCORES = 2
NSUB = 16
NW = NCORES * NSUB
EPW = E // NW
CH = 128
NCH = 80
EPAD = NCH * CH - EPW
NDUMMY = 8
RPT = 640

_sc_mesh = plsc.VectorSubcoreMesh(core_axis_name="c", subcore_axis_name="s")


@functools.partial(
    pl.kernel,
    out_type=jax.ShapeDtypeStruct((2 * N, H), jnp.float32),
    mesh=_sc_mesh,
    scratch_types=[
        pltpu.VMEM((NCH, CH), jnp.int32),
        pltpu.VMEM((CH,), jnp.int32),
        pltpu.VMEM((CH,), jnp.int32),
        pltpu.VMEM((CH,), jnp.int32),
        pltpu.VMEM((CH,), jnp.int32),
        pltpu.VMEM((CH, H), jnp.float32),
        pltpu.VMEM((CH, H), jnp.float32),
        pltpu.SemaphoreType.DMA,
        pltpu.SemaphoreType.DMA,
        pltpu.VMEM_SHARED((N + NDUMMY, H), jnp.float32),
    ],
)
def _sc_agg(h_hbm, sd_hbm, out_hbm,
            sd_v, sbuf0, sbuf1, dbuf0, dbuf1, rows0, rows1,
            gsem0, gsem1, agg_sh):
    cid = lax.axis_index("c")
    sid = lax.axis_index("s")
    wid = cid * NSUB + sid

    pltpu.sync_copy(sd_hbm.at[wid], sd_v)

    def _unpack(j, sbuf, dbuf):
        for c8 in range(CH // 16):
            v = sd_v[j, pl.ds(c8 * 16, 16)]
            sbuf[pl.ds(c8 * 16, 16)] = v & 0xFFFF
            dbuf[pl.ds(c8 * 16, 16)] = lax.shift_right_logical(v, 16)

    def _zrow(r, carry):
        for c8 in range(H // 16):
            rows0[r, pl.ds(c8 * 16, 16)] = jnp.zeros((16,), jnp.float32)
        return carry
    lax.fori_loop(0, CH, _zrow, 0)
    row0 = sid * RPT

    @pl.when(sid < NSUB - 1)
    def _():
        for t in range(RPT // CH):
            pltpu.sync_copy(rows0, agg_sh.at[pl.ds(row0 + t * CH, CH)])

    @pl.when(sid == NSUB - 1)
    def _():
        for t in range(3):
            pltpu.sync_copy(rows0, agg_sh.at[pl.ds(row0 + t * CH, CH)])
        last = N + NDUMMY - (NSUB - 1) * RPT - 3 * CH
        pltpu.sync_copy(rows0.at[pl.ds(0, last)],
                        agg_sh.at[pl.ds(row0 + 3 * CH, last)])
    plsc.subcore_barrier()

    def _chunk(j, carry):
        _unpack(j, sbuf0, dbuf0)
        pltpu.async_copy(h_hbm.at[sbuf0], rows0, gsem0).wait()
        pltpu.sync_copy(rows0, agg_sh.at[dbuf0], add=True)
        return carry
    lax.fori_loop(0, NCH, _chunk, 0)

    plsc.subcore_barrier()

    @pl.when(sid < NSUB - 1)
    def _():
        pltpu.sync_copy(agg_sh.at[pl.ds(row0, RPT)],
                        out_hbm.at[pl.ds(cid * N + row0, RPT)])

    @pl.when(sid == NSUB - 1)
    def _():
        pltpu.sync_copy(agg_sh.at[pl.ds(row0, N - (NSUB - 1) * RPT)],
                        out_hbm.at[pl.ds(cid * N + row0, N - (NSUB - 1) * RPT)])


BR = 1000
NBLK = N // BR


def _mlp_body(h_ref, a0_ref, a1_ref, w1_ref, b1_ref, w2_ref, b2_ref, o_ref):
    z = h_ref[...] + a0_ref[...] + a1_ref[...]
    t = jnp.maximum(
        jnp.dot(z, w1_ref[...], preferred_element_type=jnp.float32) + b1_ref[...],
        0.0)
    o_ref[...] = jnp.maximum(
        jnp.dot(t, w2_ref[...], preferred_element_type=jnp.float32) + b2_ref[...],
        0.0)


_mlp = pl.pallas_call(
    _mlp_body,
    grid=(NBLK,),
    in_specs=[
        pl.BlockSpec((BR, H), lambda i: (i, 0)),
        pl.BlockSpec((BR, H), lambda i: (i, 0)),
        pl.BlockSpec((BR, H), lambda i: (NBLK + i, 0)),
        pl.BlockSpec((H, H), lambda i: (0, 0)),
        pl.BlockSpec((1, H), lambda i: (0, 0)),
        pl.BlockSpec((H, H), lambda i: (0, 0)),
        pl.BlockSpec((1, H), lambda i: (0, 0)),
    ],
    out_specs=pl.BlockSpec((BR, H), lambda i: (i, 0)),
    out_shape=jax.ShapeDtypeStruct((N, H), jnp.float32),
)

_BN_SCALE = float(1.0 / np.sqrt(1.0 + 1e-5))


def _pool_cls_body(b_ref, h1_ref, h2_ref, h3_ref, cw1_ref, cb1_ref,
                   g_ref, be_ref, cw2_ref, cb2_ref, o_ref, acc_ref):
    i = pl.program_id(0)

    @pl.when(i == 0)
    def _():
        acc_ref[...] = jnp.zeros_like(acc_ref)

    oh = (b_ref[...] == lax.broadcasted_iota(jnp.int32, (1, G), 1)
          ).astype(jnp.float32)
    hcat = jnp.concatenate([h1_ref[...], h2_ref[...], h3_ref[...]], axis=1)
    acc_ref[...] += jnp.dot(oh.T, hcat, preferred_element_type=jnp.float32)

    @pl.when(i == pl.num_programs(0) - 1)
    def _():
        z = jnp.dot(acc_ref[...], cw1_ref[...],
                    preferred_element_type=jnp.float32) + cb1_ref[...]
        z = z * _BN_SCALE * g_ref[...] + be_ref[...]
        z = jnp.maximum(z, 0.0)
        o_ref[...] = jnp.dot(z, cw2_ref[...],
                             preferred_element_type=jnp.float32) + cb2_ref[...]


_pool_cls = pl.pallas_call(
    _pool_cls_body,
    grid=(NBLK,),
    in_specs=[
        pl.BlockSpec((BR, 1), lambda i: (i, 0)),
        pl.BlockSpec((BR, H), lambda i: (i, 0)),
        pl.BlockSpec((BR, H), lambda i: (i, 0)),
        pl.BlockSpec((BR, H), lambda i: (i, 0)),
        pl.BlockSpec((3 * H, 2 * H), lambda i: (0, 0)),
        pl.BlockSpec((1, 2 * H), lambda i: (0, 0)),
        pl.BlockSpec((1, 2 * H), lambda i: (0, 0)),
        pl.BlockSpec((1, 2 * H), lambda i: (0, 0)),
        pl.BlockSpec((2 * H, 128), lambda i: (0, 0)),
        pl.BlockSpec((1, 128), lambda i: (0, 0)),
    ],
    out_specs=pl.BlockSpec((G, 128), lambda i: (0, 0)),
    out_shape=jax.ShapeDtypeStruct((G, 128), jnp.float32),
    scratch_shapes=[pltpu.VMEM((G, 3 * H), jnp.float32)],
)


def kernel(x, edge_index, batch, W1_0, b1_0, W2_0, b2_0, W1_1, b1_1, W2_1,
           b2_1, W1_2, b1_2, W2_2, b2_2, cW1, cb1, bn_gamma, bn_beta, cW2,
           cb2):
    sd = edge_index[0] | (edge_index[1] << 16)
    sd_p = jnp.pad(sd.reshape(NW, EPW), ((0, 0), (0, EPAD)),
                   constant_values=N << 16).reshape(NW, NCH, CH)
    params = [(W1_0, b1_0, W2_0, b2_0), (W1_1, b1_1, W2_1, b2_1),
              (W1_2, b1_2, W2_2, b2_2)]

    h = x
    hs = []
    for (W1, b1, W2, b2) in params:
        agg = _sc_agg(h, sd_p)
        h = _mlp(h, agg, agg, W1, b1.reshape(1, H), W2, b2.reshape(1, H))
        hs.append(h)

    cW2p = jnp.zeros((2 * H, 128), jnp.float32).at[:, :NC].set(cW2)
    cb2p = jnp.zeros((1, 128), jnp.float32).at[0, :NC].set(cb2)
    out = _pool_cls(batch.reshape(N, 1), hs[0], hs[1], hs[2], cW1,
                    cb1.reshape(1, 2 * H), bn_gamma.reshape(1, 2 * H),
                    bn_beta.reshape(1, 2 * H), cW2p, cb2p)
    return out[:, :NC]


# speedup vs baseline: 1.1611x; 1.1611x over previous
import functools

import jax
import jax.numpy as jnp
import numpy as np
from jax import lax
from jax.experimental import pallas as pl
from jax.experimental.pallas import tpu as pltpu
from jax.experimental.pallas import tpu_sc as plsc

N = 10000
E = 320000
D = 128
H = 128
G = 64
NC = 2

NCORES = 2
NSUB = 16
NW = NCORES * NSUB
EPW = E // NW
CH = 128
NCH = 80
EPAD = NCH * CH - EPW
NDUMMY = 8
RPT = 640

_sc_mesh = plsc.VectorSubcoreMesh(core_axis_name="c", subcore_axis_name="s")


@functools.partial(
    pl.kernel,
    out_type=jax.ShapeDtypeStruct((2 * N, H), jnp.float32),
    mesh=_sc_mesh,
    scratch_types=[
        [pltpu.VMEM((CH,), jnp.int32) for _ in range(4)],
        [pltpu.VMEM((CH,), jnp.int32) for _ in range(4)],
        [pltpu.VMEM((CH, H), jnp.float32) for _ in range(2)],
        [pltpu.SemaphoreType.DMA for _ in range(4)],
        [pltpu.SemaphoreType.DMA for _ in range(2)],
        pltpu.VMEM_SHARED((N + NDUMMY, H), jnp.float32),
    ],
)
def _sc_agg(h_hbm, src_hbm, dst_hbm, out_hbm,
            sbufs, dbufs, rows, isems, gsems, agg_sh):
    cid = lax.axis_index("c")
    sid = lax.axis_index("s")
    wid = cid * NSUB + sid
    rows0 = rows[0]

    def _zrow(r, carry):
        for c8 in range(H // 16):
            rows0[r, pl.ds(c8 * 16, 16)] = jnp.zeros((16,), jnp.float32)
        return carry
    lax.fori_loop(0, CH, _zrow, 0)
    row0 = sid * RPT

    @pl.when(sid < NSUB - 1)
    def _():
        for t in range(RPT // CH):
            pltpu.sync_copy(rows0, agg_sh.at[pl.ds(row0 + t * CH, CH)])

    @pl.when(sid == NSUB - 1)
    def _():
        for t in range(3):
            pltpu.sync_copy(rows0, agg_sh.at[pl.ds(row0 + t * CH, CH)])
        last = N + NDUMMY - (NSUB - 1) * RPT - 3 * CH
        pltpu.sync_copy(rows0.at[pl.ds(0, last)],
                        agg_sh.at[pl.ds(row0 + 3 * CH, last)])
    plsc.subcore_barrier()

    def _idx(j, k):
        pltpu.async_copy(src_hbm.at[wid, j], sbufs[k], isems[k])
        pltpu.async_copy(dst_hbm.at[wid, j], dbufs[k], isems[k])

    def _idx_wait(j, k):
        pltpu.make_async_copy(src_hbm.at[wid, j], sbufs[k], isems[k]).wait()
        pltpu.make_async_copy(dst_hbm.at[wid, j], dbufs[k], isems[k]).wait()

    def _gather(k, g):
        pltpu.async_copy(h_hbm.at[sbufs[k]], rows[g], gsems[g])

    def _gather_wait(k, g):
        pltpu.make_async_copy(h_hbm.at[sbufs[k]], rows[g], gsems[g]).wait()

    for k in range(4):
        _idx(k, k)
    for k in range(2):
        _idx_wait(k, k)
        _gather(k, k)

    def _quad(t, carry):
        j0 = 4 * t
        for k in range(4):
            g = k % 2
            _gather_wait(k, g)
            pltpu.sync_copy(rows[g], agg_sh.at[dbufs[k]], add=True)
            _idx(j0 + k + 4, k)
            _idx_wait(j0 + k + 2, (k + 2) % 4)
            _gather((k + 2) % 4, g)
        return carry
    lax.fori_loop(0, NCH // 4 - 1, _quad, 0)

    for k in range(4):
        g = k % 2
        _gather_wait(k, g)
        pltpu.sync_copy(rows[g], agg_sh.at[dbufs[k]], add=True)
        if k < 2:
            _idx_wait(NCH - 2 + k, (k + 2) % 4)
            _gather((k + 2) % 4, g)

    plsc.subcore_barrier()

    @pl.when(sid < NSUB - 1)
    def _():
        pltpu.sync_copy(agg_sh.at[pl.ds(row0, RPT)],
                        out_hbm.at[pl.ds(cid * N + row0, RPT)])

    @pl.when(sid == NSUB - 1)
    def _():
        pltpu.sync_copy(agg_sh.at[pl.ds(row0, N - (NSUB - 1) * RPT)],
                        out_hbm.at[pl.ds(cid * N + row0, N - (NSUB - 1) * RPT)])


BR = 1000
NBLK = N // BR


def _mlp_body(h_ref, a0_ref, a1_ref, w1_ref, b1_ref, w2_ref, b2_ref, o_ref):
    z = h_ref[...] + a0_ref[...] + a1_ref[...]
    t = jnp.maximum(
        jnp.dot(z, w1_ref[...], preferred_element_type=jnp.float32) + b1_ref[...],
        0.0)
    o_ref[...] = jnp.maximum(
        jnp.dot(t, w2_ref[...], preferred_element_type=jnp.float32) + b2_ref[...],
        0.0)


_mlp = pl.pallas_call(
    _mlp_body,
    grid=(NBLK,),
    in_specs=[
        pl.BlockSpec((BR, H), lambda i: (i, 0)),
        pl.BlockSpec((BR, H), lambda i: (i, 0)),
        pl.BlockSpec((BR, H), lambda i: (NBLK + i, 0)),
        pl.BlockSpec((H, H), lambda i: (0, 0)),
        pl.BlockSpec((1, H), lambda i: (0, 0)),
        pl.BlockSpec((H, H), lambda i: (0, 0)),
        pl.BlockSpec((1, H), lambda i: (0, 0)),
    ],
    out_specs=pl.BlockSpec((BR, H), lambda i: (i, 0)),
    out_shape=jax.ShapeDtypeStruct((N, H), jnp.float32),
)

_BN_SCALE = float(1.0 / np.sqrt(1.0 + 1e-5))


def _pool_cls_body(b_ref, h1_ref, h2_ref, h3_ref, cw1_ref, cb1_ref,
                   g_ref, be_ref, cw2_ref, cb2_ref, o_ref, acc_ref):
    i = pl.program_id(0)

    @pl.when(i == 0)
    def _():
        acc_ref[...] = jnp.zeros_like(acc_ref)

    oh = (b_ref[...] == lax.broadcasted_iota(jnp.int32, (1, G), 1)
          ).astype(jnp.float32)
    hcat = jnp.concatenate([h1_ref[...], h2_ref[...], h3_ref[...]], axis=1)
    acc_ref[...] += jnp.dot(oh.T, hcat, preferred_element_type=jnp.float32)

    @pl.when(i == pl.num_programs(0) - 1)
    def _():
        z = jnp.dot(acc_ref[...], cw1_ref[...],
                    preferred_element_type=jnp.float32) + cb1_ref[...]
        z = z * _BN_SCALE * g_ref[...] + be_ref[...]
        z = jnp.maximum(z, 0.0)
        o_ref[...] = jnp.dot(z, cw2_ref[...],
                             preferred_element_type=jnp.float32) + cb2_ref[...]


_pool_cls = pl.pallas_call(
    _pool_cls_body,
    grid=(NBLK,),
    in_specs=[
        pl.BlockSpec((BR, 1), lambda i: (i, 0)),
        pl.BlockSpec((BR, H), lambda i: (i, 0)),
        pl.BlockSpec((BR, H), lambda i: (i, 0)),
        pl.BlockSpec((BR, H), lambda i: (i, 0)),
        pl.BlockSpec((3 * H, 2 * H), lambda i: (0, 0)),
        pl.BlockSpec((1, 2 * H), lambda i: (0, 0)),
        pl.BlockSpec((1, 2 * H), lambda i: (0, 0)),
        pl.BlockSpec((1, 2 * H), lambda i: (0, 0)),
        pl.BlockSpec((2 * H, 128), lambda i: (0, 0)),
        pl.BlockSpec((1, 128), lambda i: (0, 0)),
    ],
    out_specs=pl.BlockSpec((G, 128), lambda i: (0, 0)),
    out_shape=jax.ShapeDtypeStruct((G, 128), jnp.float32),
    scratch_shapes=[pltpu.VMEM((G, 3 * H), jnp.float32)],
)


def kernel(x, edge_index, batch, W1_0, b1_0, W2_0, b2_0, W1_1, b1_1, W2_1,
           b2_1, W1_2, b1_2, W2_2, b2_2, cW1, cb1, bn_gamma, bn_beta, cW2,
           cb2):
    src_p = jnp.pad(edge_index[0].reshape(NW, EPW),
                    ((0, 0), (0, EPAD))).reshape(NW, NCH, CH)
    dst_p = jnp.pad(edge_index[1].reshape(NW, EPW),
                    ((0, 0), (0, EPAD)),
                    constant_values=N).reshape(NW, NCH, CH)
    params = [(W1_0, b1_0, W2_0, b2_0), (W1_1, b1_1, W2_1, b2_1),
              (W1_2, b1_2, W2_2, b2_2)]

    h = x
    hs = []
    for (W1, b1, W2, b2) in params:
        agg = _sc_agg(h, src_p, dst_p)
        h = _mlp(h, agg, agg, W1, b1.reshape(1, H), W2, b2.reshape(1, H))
        hs.append(h)

    cW2p = jnp.zeros((2 * H, 128), jnp.float32).at[:, :NC].set(cW2)
    cb2p = jnp.zeros((1, 128), jnp.float32).at[0, :NC].set(cb2)
    out = _pool_cls(batch.reshape(N, 1), hs[0], hs[1], hs[2], cW1,
                    cb1.reshape(1, 2 * H), bn_gamma.reshape(1, 2 * H),
                    bn_beta.reshape(1, 2 * H), cW2p, cb2p)
    return out[:, :NC]
